# Initial kernel scaffold; baseline (speedup 1.0000x reference)
import functools

import numpy as np
import jax
import jax.numpy as jnp
from jax import lax
from jax.experimental import pallas as pl
from jax.experimental.pallas import tpu as pltpu
from jax.experimental.pallas import tpu_sc as plsc

DIM = 768
E = 16
NFRQ = 2048
B = 64
ALPHA = 300.0
NSQ = DIM * DIM

HALF = NSQ // 2
ZCH = 49152
NCH = NFRQ // 2 // 128


def _trig_tables():
    m = np.arange(DIM)
    mj = np.outer(m, m) % DIM
    ang = (2.0 * np.pi / DIM) * mj
    scale = np.sqrt(ALPHA) / DIM
    cos = (np.cos(ang) * scale).astype(np.float32)
    sin = (np.sin(ang) * scale).astype(np.float32)
    return cos, sin


_COS_TAB, _SIN_TAB = _trig_tables()


def _scatter_body(coeff_hbm, idx_hbm, t_hbm, zbuf, vals_v, idx_v, sem):
    c = lax.axis_index("c")
    s = lax.axis_index("s")
    e = c * (E // 2) + s // 2
    h = s % 2

    def zfill(i, carry):
        zbuf[pl.ds(i * 16, 16)] = jnp.zeros((16,), jnp.float32)
        return carry

    lax.fori_loop(0, ZCH // 16, zfill, 0)

    def zcopy(i, carry):
        pltpu.sync_copy(zbuf, t_hbm.at[e, pl.ds(h * HALF + i * ZCH, ZCH)])
        return carry

    lax.fori_loop(0, HALF // ZCH, zcopy, 0)

    plsc.subcore_barrier()

    pltpu.sync_copy(coeff_hbm.at[e, h], vals_v)
    pltpu.sync_copy(idx_hbm.at[e, h], idx_v)
    for j in range(NCH):
        pltpu.async_copy(vals_v.at[j], t_hbm.at[e].at[idx_v.at[j]], sem).wait()


_scatter = functools.partial(
    pl.kernel,
    mesh=plsc.VectorSubcoreMesh(core_axis_name="c", subcore_axis_name="s"),
    out_type=jax.ShapeDtypeStruct((E, NSQ), jnp.float32),
    scratch_types=[
        pltpu.VMEM((ZCH,), jnp.float32),
        pltpu.VMEM((NCH, 128), jnp.float32),
        pltpu.VMEM((NCH, 128), jnp.int32),
        pltpu.SemaphoreType.DMA,
    ],
)(_scatter_body)


def _dft_body(t_ref, c_ref, s_ref, z_ref):
    t = t_ref[0]
    ct = c_ref[...]
    st = s_ref[...]
    p = jnp.dot(t, ct, preferred_element_type=jnp.float32)
    q = jnp.dot(t, st, preferred_element_type=jnp.float32)
    z_ref[0] = (
        jnp.dot(ct, p, preferred_element_type=jnp.float32)
        - jnp.dot(st, q, preferred_element_type=jnp.float32)
    )


def _dft(t3):
    return pl.pallas_call(
        _dft_body,
        grid=(E,),
        in_specs=[
            pl.BlockSpec((1, DIM, DIM), lambda i: (i, 0, 0)),
            pl.BlockSpec((DIM, DIM), lambda i: (0, 0)),
            pl.BlockSpec((DIM, DIM), lambda i: (0, 0)),
        ],
        out_specs=pl.BlockSpec((1, DIM, DIM), lambda i: (i, 0, 0)),
        out_shape=jax.ShapeDtypeStruct((E, DIM, DIM), jnp.float32),
    )(t3, jnp.asarray(_COS_TAB), jnp.asarray(_SIN_TAB))


NT = 8
CW = NSQ // NT


def _combine_body(cls_ref, w_ref, b_ref, z_ref, o_ref):
    logits = (
        jnp.dot(cls_ref[...], w_ref[...], preferred_element_type=jnp.float32)
        + b_ref[...]
    )
    iota = lax.broadcasted_iota(jnp.int32, (B, E), 1)
    big = jnp.int32(2 * E)
    mx1 = jnp.max(logits, axis=1, keepdims=True)
    i1 = jnp.min(jnp.where(logits == mx1, iota, big), axis=1, keepdims=True)
    m1 = iota == i1
    masked = jnp.where(m1, jnp.float32(-1e30), logits)
    mx2 = jnp.max(masked, axis=1, keepdims=True)
    i2 = jnp.min(jnp.where(masked == mx2, iota, big), axis=1, keepdims=True)
    m = m1 | (iota == i2)
    o_ref[...] = jnp.dot(
        m.astype(jnp.float32), z_ref[...], preferred_element_type=jnp.float32
    )


def _combine(cls_token, router_W, router_b2, z2):
    return pl.pallas_call(
        _combine_body,
        grid=(NT,),
        in_specs=[
            pl.BlockSpec((B, DIM), lambda i: (0, 0)),
            pl.BlockSpec((DIM, E), lambda i: (0, 0)),
            pl.BlockSpec((1, E), lambda i: (0, 0)),
            pl.BlockSpec((E, CW), lambda i: (0, i)),
        ],
        out_specs=pl.BlockSpec((B, CW), lambda i: (0, i)),
        out_shape=jax.ShapeDtypeStruct((B, NSQ), jnp.float32),
    )(cls_token, router_W, router_b2, z2)


def kernel(cls_token, router_W, router_b, coeff, list_indices):
    coeff_r = coeff.reshape(E, 2, NCH, 128)
    idx_r = list_indices.reshape(E, 2, NCH, 128)
    t = _scatter(coeff_r, idx_r)
    z = _dft(t.reshape(E, DIM, DIM))
    out2 = _combine(
        cls_token, router_W, router_b.reshape(1, E), z.reshape(E, NSQ)
    )
    return out2.reshape(B, DIM, DIM)


# speedup vs baseline: 11.3501x; 11.3501x over previous
import functools

import numpy as np
import jax
import jax.numpy as jnp
from jax import lax
from jax.experimental import pallas as pl
from jax.experimental.pallas import tpu as pltpu
from jax.experimental.pallas import tpu_sc as plsc

DIM = 768
E = 16
NFRQ = 2048
B = 64
ALPHA = 300.0
NSQ = DIM * DIM

def _trig_tables():
    m = np.arange(DIM)
    mj = np.outer(m, m) % DIM
    ang = (2.0 * np.pi / DIM) * mj
    scale = np.sqrt(ALPHA) / DIM
    cos = (np.cos(ang) * scale).astype(np.float32)
    sin = (np.sin(ang) * scale).astype(np.float32)
    return cos, sin


_COS_TAB, _SIN_TAB = _trig_tables()


EPC = E // 2
SLC = NSQ // 16


def _scatter_body(coeff_hbm, idx_hbm, t_hbm, buf, vals_v, idx_v):
    c = lax.axis_index("c")
    s = lax.axis_index("s")
    lo = s * SLC

    def per_expert(eo, carry):
        e = c * EPC + eo

        def zfill(i, cc):
            buf[pl.ds(i * 16, 16)] = jnp.zeros((16,), jnp.float32)
            return cc

        lax.fori_loop(0, SLC // 16, zfill, 0)

        pltpu.sync_copy(coeff_hbm.at[e], vals_v)
        pltpu.sync_copy(idx_hbm.at[e], idx_v)

        def scat(v, cc):
            idx = idx_v[pl.ds(v * 16, 16)]
            val = vals_v[pl.ds(v * 16, 16)]
            rel = idx - lo
            mask = (rel >= 0) & (rel < SLC)
            rel = jnp.where(mask, rel, 0)
            plsc.store_scatter(buf, [rel], val, mask=mask)
            return cc

        lax.fori_loop(0, NFRQ // 16, scat, 0)
        pltpu.sync_copy(buf, t_hbm.at[e, pl.ds(lo, SLC)])
        return carry

    lax.fori_loop(0, EPC, per_expert, 0)


def _scatter(coeff, list_indices):
    run = functools.partial(
        pl.kernel,
        mesh=plsc.VectorSubcoreMesh(core_axis_name="c", subcore_axis_name="s"),
        compiler_params=pltpu.CompilerParams(needs_layout_passes=False),
        out_type=jax.ShapeDtypeStruct((E, NSQ), jnp.float32),
        scratch_types=[
            pltpu.VMEM((SLC,), jnp.float32),
            pltpu.VMEM((NFRQ,), jnp.float32),
            pltpu.VMEM((NFRQ,), jnp.int32),
        ],
    )(_scatter_body)
    return run(coeff, list_indices)


def _dft_body(t_ref, c_ref, s_ref, z_ref):
    t = t_ref[0]
    ct = c_ref[...]
    st = s_ref[...]
    p = jnp.dot(t, ct, preferred_element_type=jnp.float32)
    q = jnp.dot(t, st, preferred_element_type=jnp.float32)
    z_ref[0] = (
        jnp.dot(ct, p, preferred_element_type=jnp.float32)
        - jnp.dot(st, q, preferred_element_type=jnp.float32)
    )


def _dft(t3):
    return pl.pallas_call(
        _dft_body,
        grid=(E,),
        in_specs=[
            pl.BlockSpec((1, DIM, DIM), lambda i: (i, 0, 0)),
            pl.BlockSpec((DIM, DIM), lambda i: (0, 0)),
            pl.BlockSpec((DIM, DIM), lambda i: (0, 0)),
        ],
        out_specs=pl.BlockSpec((1, DIM, DIM), lambda i: (i, 0, 0)),
        out_shape=jax.ShapeDtypeStruct((E, DIM, DIM), jnp.float32),
    )(t3, jnp.asarray(_COS_TAB), jnp.asarray(_SIN_TAB))


NT = 8
CW = NSQ // NT


def _combine_body(cls_ref, w_ref, b_ref, z_ref, o_ref):
    logits = (
        jnp.dot(cls_ref[...], w_ref[...], preferred_element_type=jnp.float32)
        + b_ref[...]
    )
    iota = lax.broadcasted_iota(jnp.int32, (B, E), 1)
    big = jnp.int32(2 * E)
    mx1 = jnp.max(logits, axis=1, keepdims=True)
    i1 = jnp.min(jnp.where(logits == mx1, iota, big), axis=1, keepdims=True)
    m1 = iota == i1
    masked = jnp.where(m1, jnp.float32(-1e30), logits)
    mx2 = jnp.max(masked, axis=1, keepdims=True)
    i2 = jnp.min(jnp.where(masked == mx2, iota, big), axis=1, keepdims=True)
    m = m1 | (iota == i2)
    o_ref[...] = jnp.dot(
        m.astype(jnp.float32), z_ref[...], preferred_element_type=jnp.float32
    )


def _combine(cls_token, router_W, router_b2, z2):
    return pl.pallas_call(
        _combine_body,
        grid=(NT,),
        in_specs=[
            pl.BlockSpec((B, DIM), lambda i: (0, 0)),
            pl.BlockSpec((DIM, E), lambda i: (0, 0)),
            pl.BlockSpec((1, E), lambda i: (0, 0)),
            pl.BlockSpec((E, CW), lambda i: (0, i)),
        ],
        out_specs=pl.BlockSpec((B, CW), lambda i: (0, i)),
        out_shape=jax.ShapeDtypeStruct((B, NSQ), jnp.float32),
    )(cls_token, router_W, router_b2, z2)


def kernel(cls_token, router_W, router_b, coeff, list_indices):
    t = _scatter(coeff, list_indices)
    z = _dft(t.reshape(E, DIM, DIM))
    out2 = _combine(
        cls_token, router_W, router_b.reshape(1, E), z.reshape(E, NSQ)
    )
    return out2.reshape(B, DIM, DIM)


# speedup vs baseline: 16.7994x; 1.4801x over previous
import functools

import numpy as np
import jax
import jax.numpy as jnp
from jax import lax
from jax.experimental import pallas as pl
from jax.experimental.pallas import tpu as pltpu
from jax.experimental.pallas import tpu_sc as plsc

DIM = 768
E = 16
NFRQ = 2048
B = 64
ALPHA = 300.0
NSQ = DIM * DIM

def _trig_tables():
    m = np.arange(DIM)
    mj = np.outer(m, m) % DIM
    ang = (2.0 * np.pi / DIM) * mj
    scale = np.sqrt(ALPHA) / DIM
    cos = (np.cos(ang) * scale).astype(np.float32)
    sin = (np.sin(ang) * scale).astype(np.float32)
    return cos, sin


_COS_TAB, _SIN_TAB = _trig_tables()


EPC = E // 2
SLR = DIM // 16


def _scatter_body(coeff_hbm, idx_hbm, t_hbm, buf, vals_v, idx_v):
    c = lax.axis_index("c")
    s = lax.axis_index("s")
    row0 = s * SLR

    def zfill(i, cc):
        buf[i // (DIM // 16), pl.ds((i % (DIM // 16)) * 16, 16)] = jnp.zeros(
            (16,), jnp.float32
        )
        return cc

    lax.fori_loop(0, SLR * DIM // 16, zfill, 0, unroll=8)

    def local_coords(v):
        idx = idx_v[pl.ds(v * 16, 16)]
        row = idx // DIM
        col = idx - row * DIM
        rr = row - row0
        mask = (rr >= 0) & (rr < SLR)
        rr = jnp.where(mask, rr, 0)
        col = jnp.where(mask, col, 0)
        return rr, col, mask

    def per_expert(eo, carry):
        e = c * EPC + eo
        pltpu.sync_copy(coeff_hbm.at[e], vals_v)
        pltpu.sync_copy(idx_hbm.at[e], idx_v)

        def scat(v, cc):
            rr, col, mask = local_coords(v)
            val = vals_v[pl.ds(v * 16, 16)]
            plsc.store_scatter(buf, [rr, col], val, mask=mask)
            return cc

        lax.fori_loop(0, NFRQ // 16, scat, 0, unroll=4)
        pltpu.sync_copy(buf, t_hbm.at[e, pl.ds(row0, SLR), :])

        def unscat(v, cc):
            rr, col, mask = local_coords(v)
            plsc.store_scatter(
                buf, [rr, col], jnp.zeros((16,), jnp.float32), mask=mask
            )
            return cc

        lax.fori_loop(0, NFRQ // 16, unscat, 0, unroll=4)
        return carry

    lax.fori_loop(0, EPC, per_expert, 0)


def _scatter(coeff, list_indices):
    run = functools.partial(
        pl.kernel,
        mesh=plsc.VectorSubcoreMesh(core_axis_name="c", subcore_axis_name="s"),
        compiler_params=pltpu.CompilerParams(needs_layout_passes=False),
        out_type=jax.ShapeDtypeStruct((E, DIM, DIM), jnp.float32),
        scratch_types=[
            pltpu.VMEM((SLR, DIM), jnp.float32),
            pltpu.VMEM((NFRQ,), jnp.float32),
            pltpu.VMEM((NFRQ,), jnp.int32),
        ],
    )(_scatter_body)
    return run(coeff, list_indices)


def _dft_body(t_ref, c_ref, s_ref, z_ref):
    t = t_ref[0]
    ct = c_ref[...]
    st = s_ref[...]
    p = jnp.dot(t, ct, preferred_element_type=jnp.float32)
    q = jnp.dot(t, st, preferred_element_type=jnp.float32)
    z_ref[0] = (
        jnp.dot(ct, p, preferred_element_type=jnp.float32)
        - jnp.dot(st, q, preferred_element_type=jnp.float32)
    )


def _dft(t3):
    return pl.pallas_call(
        _dft_body,
        grid=(E,),
        in_specs=[
            pl.BlockSpec((1, DIM, DIM), lambda i: (i, 0, 0)),
            pl.BlockSpec((DIM, DIM), lambda i: (0, 0)),
            pl.BlockSpec((DIM, DIM), lambda i: (0, 0)),
        ],
        out_specs=pl.BlockSpec((1, DIM, DIM), lambda i: (i, 0, 0)),
        out_shape=jax.ShapeDtypeStruct((E, DIM, DIM), jnp.float32),
    )(t3, jnp.asarray(_COS_TAB), jnp.asarray(_SIN_TAB))


NT = 8
RT = DIM // NT


def _combine_body(cls_ref, w_ref, b_ref, z_ref, o_ref):
    logits = (
        jnp.dot(cls_ref[...], w_ref[...], preferred_element_type=jnp.float32)
        + b_ref[...]
    )
    iota = lax.broadcasted_iota(jnp.int32, (B, E), 1)
    big = jnp.int32(2 * E)
    mx1 = jnp.max(logits, axis=1, keepdims=True)
    i1 = jnp.min(jnp.where(logits == mx1, iota, big), axis=1, keepdims=True)
    m1 = iota == i1
    masked = jnp.where(m1, jnp.float32(-1e30), logits)
    mx2 = jnp.max(masked, axis=1, keepdims=True)
    i2 = jnp.min(jnp.where(masked == mx2, iota, big), axis=1, keepdims=True)
    m = m1 | (iota == i2)
    o_ref[...] = lax.dot_general(
        m.astype(jnp.float32),
        z_ref[...],
        (((1,), (0,)), ((), ())),
        preferred_element_type=jnp.float32,
    )


def _combine(cls_token, router_W, router_b2, z):
    return pl.pallas_call(
        _combine_body,
        grid=(NT,),
        in_specs=[
            pl.BlockSpec((B, DIM), lambda i: (0, 0)),
            pl.BlockSpec((DIM, E), lambda i: (0, 0)),
            pl.BlockSpec((1, E), lambda i: (0, 0)),
            pl.BlockSpec((E, RT, DIM), lambda i: (0, i, 0)),
        ],
        out_specs=pl.BlockSpec((B, RT, DIM), lambda i: (0, i, 0)),
        out_shape=jax.ShapeDtypeStruct((B, DIM, DIM), jnp.float32),
    )(cls_token, router_W, router_b2, z)


def kernel(cls_token, router_W, router_b, coeff, list_indices):
    t = _scatter(coeff, list_indices)
    z = _dft(t)
    return _combine(cls_token, router_W, router_b.reshape(1, E), z)


# speedup vs baseline: 18.4717x; 1.0995x over previous
import functools

import numpy as np
import jax
import jax.numpy as jnp
from jax import lax
from jax.experimental import pallas as pl
from jax.experimental.pallas import tpu as pltpu
from jax.experimental.pallas import tpu_sc as plsc

DIM = 768
E = 16
NFRQ = 2048
B = 64
ALPHA = 300.0
NSQ = DIM * DIM

def _trig_tables():
    m = np.arange(DIM)
    mj = np.outer(m, m) % DIM
    ang = (2.0 * np.pi / DIM) * mj
    scale = np.sqrt(ALPHA) / DIM
    cos = (np.cos(ang) * scale).astype(np.float32)
    sin = (np.sin(ang) * scale).astype(np.float32)
    return cos, sin


_COS_TAB, _SIN_TAB = _trig_tables()


EPC = E // 2
SLR = DIM // 16


def _scatter_body(coeff_hbm, idx_hbm, t_hbm, buf0, buf1, vals_all, idx_all,
                  sem0, sem1):
    c = lax.axis_index("c")
    s = lax.axis_index("s")
    row0 = s * SLR

    pltpu.sync_copy(coeff_hbm.at[pl.ds(c * EPC, EPC)], vals_all)
    pltpu.sync_copy(idx_hbm.at[pl.ds(c * EPC, EPC)], idx_all)

    bufs = (buf0, buf1)
    sems = (sem0, sem1)

    def zero_buf(buf):
        def zfill(i, cc):
            buf[i // (DIM // 16), pl.ds((i % (DIM // 16)) * 16, 16)] = (
                jnp.zeros((16,), jnp.float32)
            )
            return cc

        lax.fori_loop(0, SLR * DIM // 16, zfill, 0, unroll=8)

    zero_buf(buf0)
    zero_buf(buf1)

    pending = [None, None]
    for eo in range(EPC):
        b = eo % 2
        buf = bufs[b]
        if pending[b] is not None:
            pending[b].wait()
            zero_buf(buf)

        def scat(v, cc, eo=eo, buf=buf):
            idx = idx_all[eo, pl.ds(v * 16, 16)]
            val = vals_all[eo, pl.ds(v * 16, 16)]
            row = idx // DIM
            col = idx - row * DIM
            rr = row - row0
            mask = (rr >= 0) & (rr < SLR)
            rr = jnp.where(mask, rr, 0)
            col = jnp.where(mask, col, 0)
            plsc.store_scatter(buf, [rr, col], val, mask=mask)
            return cc

        lax.fori_loop(0, NFRQ // 16, scat, 0, unroll=4)
        pending[b] = pltpu.async_copy(
            buf, t_hbm.at[c * EPC + eo, pl.ds(row0, SLR), :], sems[b]
        )
    for p in pending:
        p.wait()


def _scatter(coeff, list_indices):
    run = functools.partial(
        pl.kernel,
        mesh=plsc.VectorSubcoreMesh(core_axis_name="c", subcore_axis_name="s"),
        compiler_params=pltpu.CompilerParams(needs_layout_passes=False),
        out_type=jax.ShapeDtypeStruct((E, DIM, DIM), jnp.float32),
        scratch_types=[
            pltpu.VMEM((SLR, DIM), jnp.float32),
            pltpu.VMEM((SLR, DIM), jnp.float32),
            pltpu.VMEM((EPC, NFRQ), jnp.float32),
            pltpu.VMEM((EPC, NFRQ), jnp.int32),
            pltpu.SemaphoreType.DMA,
            pltpu.SemaphoreType.DMA,
        ],
    )(_scatter_body)
    return run(coeff, list_indices)


def _dft_body(t_ref, c_ref, s_ref, z_ref):
    t = t_ref[0]
    ct = c_ref[...]
    st = s_ref[...]
    p = jnp.dot(t, ct, preferred_element_type=jnp.float32)
    q = jnp.dot(t, st, preferred_element_type=jnp.float32)
    z_ref[0] = (
        jnp.dot(ct, p, preferred_element_type=jnp.float32)
        - jnp.dot(st, q, preferred_element_type=jnp.float32)
    )


def _dft(t3):
    return pl.pallas_call(
        _dft_body,
        grid=(E,),
        in_specs=[
            pl.BlockSpec((1, DIM, DIM), lambda i: (i, 0, 0)),
            pl.BlockSpec((DIM, DIM), lambda i: (0, 0)),
            pl.BlockSpec((DIM, DIM), lambda i: (0, 0)),
        ],
        out_specs=pl.BlockSpec((1, DIM, DIM), lambda i: (i, 0, 0)),
        out_shape=jax.ShapeDtypeStruct((E, DIM, DIM), jnp.float32),
    )(t3, jnp.asarray(_COS_TAB), jnp.asarray(_SIN_TAB))


NT = 8
RT = DIM // NT


def _combine_body(cls_ref, w_ref, b_ref, z_ref, o_ref):
    logits = (
        jnp.dot(cls_ref[...], w_ref[...], preferred_element_type=jnp.float32)
        + b_ref[...]
    )
    iota = lax.broadcasted_iota(jnp.int32, (B, E), 1)
    big = jnp.int32(2 * E)
    mx1 = jnp.max(logits, axis=1, keepdims=True)
    i1 = jnp.min(jnp.where(logits == mx1, iota, big), axis=1, keepdims=True)
    m1 = iota == i1
    masked = jnp.where(m1, jnp.float32(-1e30), logits)
    mx2 = jnp.max(masked, axis=1, keepdims=True)
    i2 = jnp.min(jnp.where(masked == mx2, iota, big), axis=1, keepdims=True)
    m = m1 | (iota == i2)
    o_ref[...] = lax.dot_general(
        m.astype(jnp.float32),
        z_ref[...],
        (((1,), (0,)), ((), ())),
        preferred_element_type=jnp.float32,
    )


def _combine(cls_token, router_W, router_b2, z):
    return pl.pallas_call(
        _combine_body,
        grid=(NT,),
        in_specs=[
            pl.BlockSpec((B, DIM), lambda i: (0, 0)),
            pl.BlockSpec((DIM, E), lambda i: (0, 0)),
            pl.BlockSpec((1, E), lambda i: (0, 0)),
            pl.BlockSpec((E, RT, DIM), lambda i: (0, i, 0)),
        ],
        out_specs=pl.BlockSpec((B, RT, DIM), lambda i: (0, i, 0)),
        out_shape=jax.ShapeDtypeStruct((B, DIM, DIM), jnp.float32),
    )(cls_token, router_W, router_b2, z)


def kernel(cls_token, router_W, router_b, coeff, list_indices):
    t = _scatter(coeff, list_indices)
    z = _dft(t)
    return _combine(cls_token, router_W, router_b.reshape(1, E), z)
